# baseline (speedup 1.0000x reference)
import jax
import jax.numpy as jnp
from jax import lax
from jax.experimental import pallas as pl
from jax.experimental.pallas import tpu as pltpu
from jax.experimental.pallas import tpu_sc as plsc

B = 16384
D_IN = 128
D_OUT = D_IN + 6
NC, NS, L = 2, 16, 16
NW = NC * NS
RW = B // NW
CH = 128
NCHUNK = RW // CH
GRP = CH // L
INV_D = 1.0 / D_OUT


def _rsqrt_nr(v):
    i = lax.bitcast_convert_type(v, jnp.int32)
    i = jnp.int32(0x5F3759DF) - lax.shift_right_logical(i, 1)
    y = lax.bitcast_convert_type(i, jnp.float32)
    for _ in range(3):
        y = y * (1.5 - 0.5 * v * y * y)
    return y


def _body(x_hbm, gb_hbm, tab_hbm, out_hbm, x_v, o_v, gb_v, tab_v):
    wid = lax.axis_index("s") * NC + lax.axis_index("c")
    pltpu.sync_copy(tab_hbm, tab_v)
    pltpu.sync_copy(gb_hbm, gb_v)
    lanes = lax.iota(jnp.int32, L)
    lx = lanes * D_IN
    lo = lanes * D_OUT

    def chunk_body(ci, carry):
        row0 = wid * RW + ci * CH
        pltpu.sync_copy(x_hbm.at[pl.ds(row0 * D_IN, CH * D_IN)], x_v)

        def grp(g, carry2):
            xb = lx + g * (L * D_IN)
            ob = lo + g * (L * D_OUT)
            v0 = plsc.load_gather(x_v, [xb])
            ei = v0.astype(jnp.int32) + 1
            ti = ei * 7
            s = jnp.zeros((L,), jnp.float32)
            q = jnp.zeros((L,), jnp.float32)
            embs = []
            for e in range(7):
                ev = plsc.load_gather(tab_v, [ti + e])
                embs.append(ev)
                s = s + ev
                q = q + ev * ev
            for c in range(1, D_IN):
                v = plsc.load_gather(x_v, [xb + c])
                s = s + v
                q = q + v * v
            mean = s * INV_D
            var = q * INV_D - mean * mean
            rstd = _rsqrt_nr(var + 1e-12)
            for e in range(7):
                t = (embs[e] - mean) * rstd
                ov = t * gb_v[pl.ds(L * e, L)] + gb_v[pl.ds(L * (D_OUT + e), L)]
                plsc.store_scatter(o_v, [ob + e], ov)
            for c in range(1, D_IN):
                v = plsc.load_gather(x_v, [xb + c])
                k = c + 6
                t = (v - mean) * rstd
                ov = t * gb_v[pl.ds(L * k, L)] + gb_v[pl.ds(L * (D_OUT + k), L)]
                plsc.store_scatter(o_v, [ob + k], ov)
            return carry2

        lax.fori_loop(0, GRP, grp, 0)
        pltpu.sync_copy(o_v, out_hbm.at[pl.ds(row0 * D_OUT, CH * D_OUT)])
        return carry

    lax.fori_loop(0, NCHUNK, chunk_body, 0)


def kernel(x, table, gamma, beta):
    gb = jnp.concatenate([gamma, beta]).astype(jnp.float32)
    gb = jnp.broadcast_to(gb[:, None], (2 * D_OUT, L)).reshape(-1)
    tab = jnp.pad(table.astype(jnp.float32).reshape(-1), (0, 7))
    mesh = plsc.VectorSubcoreMesh(core_axis_name="c", subcore_axis_name="s")
    f = pl.kernel(
        _body,
        out_type=jax.ShapeDtypeStruct((B * D_OUT,), jnp.float32),
        mesh=mesh,
        compiler_params=pltpu.CompilerParams(needs_layout_passes=False),
        scratch_types=[
            pltpu.VMEM((CH * D_IN,), jnp.float32),
            pltpu.VMEM((CH * D_OUT,), jnp.float32),
            pltpu.VMEM((2 * D_OUT * L,), jnp.float32),
            pltpu.VMEM((56,), jnp.float32),
        ],
    )
    out = f(x.reshape(-1), gb, tab)
    return out.reshape(B, D_OUT)


# speedup vs baseline: 1.4194x; 1.4194x over previous
import jax
import jax.numpy as jnp
from jax import lax
from jax.experimental import pallas as pl
from jax.experimental.pallas import tpu as pltpu
from jax.experimental.pallas import tpu_sc as plsc

B = 16384
D_IN = 128
D_OUT = D_IN + 6
NC, NS, L = 2, 16, 16
NW = NC * NS
RW = B // NW
CH = 128
NCHUNK = RW // CH
GRP = CH // L
INV_D = 1.0 / D_OUT


def _rsqrt_nr(v):
    i = lax.bitcast_convert_type(v, jnp.int32)
    i = jnp.int32(0x5F3759DF) - lax.shift_right_logical(i, 1)
    y = lax.bitcast_convert_type(i, jnp.float32)
    for _ in range(3):
        y = y * (1.5 - 0.5 * v * y * y)
    return y


def _body(x_hbm, gb_hbm, tab_hbm, out_hbm, x_v, o_v, gb_v, tab_v):
    wid = lax.axis_index("s") * NC + lax.axis_index("c")
    pltpu.sync_copy(tab_hbm, tab_v)
    pltpu.sync_copy(gb_hbm, gb_v)
    lanes = lax.iota(jnp.int32, L)
    lx = lanes * D_IN
    lo = lanes * D_OUT

    def chunk_body(ci, carry):
        row0 = wid * RW + ci * CH
        pltpu.sync_copy(x_hbm.at[pl.ds(row0 * D_IN, CH * D_IN)], x_v)

        def grp(g, carry2):
            xb = lx + g * (L * D_IN)
            ob = lo + g * (L * D_OUT)
            v0 = plsc.load_gather(x_v, [xb])
            ei = v0.astype(jnp.int32) + 1
            ti = ei * 7
            s = jnp.zeros((L,), jnp.float32)
            q = jnp.zeros((L,), jnp.float32)
            embs = []
            for e in range(7):
                ev = plsc.load_gather(tab_v, [ti + e])
                embs.append(ev)
                s = s + ev
                q = q + ev * ev

            @plsc.parallel_loop(1, D_IN, unroll=8, carry=(s, q))
            def pass1(c, sq):
                s1, q1 = sq
                v = plsc.load_gather(x_v, [xb + c])
                return s1 + v, q1 + v * v

            s, q = pass1
            mean = s * INV_D
            var = q * INV_D - mean * mean
            rstd = _rsqrt_nr(var + 1e-12)
            for e in range(7):
                t = (embs[e] - mean) * rstd
                ov = t * gb_v[pl.ds(L * e, L)] + gb_v[pl.ds(L * (D_OUT + e), L)]
                plsc.store_scatter(o_v, [ob + e], ov)

            @plsc.parallel_loop(1, D_IN, unroll=8)
            def pass2(c):
                v = plsc.load_gather(x_v, [xb + c])
                t = (v - mean) * rstd
                ov = t * gb_v[pl.ds(L * (c + 6), L)] + gb_v[pl.ds(L * (D_OUT + c + 6), L)]
                plsc.store_scatter(o_v, [ob + c + 6], ov)

            return carry2

        lax.fori_loop(0, GRP, grp, 0)
        pltpu.sync_copy(o_v, out_hbm.at[pl.ds(row0 * D_OUT, CH * D_OUT)])
        return carry

    lax.fori_loop(0, NCHUNK, chunk_body, 0)


def kernel(x, table, gamma, beta):
    gb = jnp.concatenate([gamma, beta]).astype(jnp.float32)
    gb = jnp.broadcast_to(gb[:, None], (2 * D_OUT, L)).reshape(-1)
    tab = jnp.pad(table.astype(jnp.float32).reshape(-1), (0, 7))
    mesh = plsc.VectorSubcoreMesh(core_axis_name="c", subcore_axis_name="s")
    f = pl.kernel(
        _body,
        out_type=jax.ShapeDtypeStruct((B * D_OUT,), jnp.float32),
        mesh=mesh,
        compiler_params=pltpu.CompilerParams(needs_layout_passes=False),
        scratch_types=[
            pltpu.VMEM((CH * D_IN,), jnp.float32),
            pltpu.VMEM((CH * D_OUT,), jnp.float32),
            pltpu.VMEM((2 * D_OUT * L,), jnp.float32),
            pltpu.VMEM((56,), jnp.float32),
        ],
    )
    out = f(x.reshape(-1), gb, tab)
    return out.reshape(B, D_OUT)


# speedup vs baseline: 3.2166x; 2.2662x over previous
import jax
import jax.numpy as jnp
from jax import lax
from jax.experimental import pallas as pl
from jax.experimental.pallas import tpu as pltpu
from jax.experimental.pallas import tpu_sc as plsc

B = 16384
D_IN = 128
D_OUT = D_IN + 6
NC, NS, L = 2, 16, 16
NW = NC * NS
RW = B // NW
CH = 128
NCHUNK = RW // CH
INV_D = 1.0 / D_OUT


def _rsqrt_nr(v):
    i = lax.bitcast_convert_type(v, jnp.int32)
    i = jnp.int32(0x5F3759DF) - lax.shift_right_logical(i, 1)
    y = lax.bitcast_convert_type(i, jnp.float32)
    for _ in range(3):
        y = y * (1.5 - 0.5 * v * y * y)
    return y


def _body(x_hbm, gb_hbm, tab_hbm, out_hbm, x_v, o_v, gb_v, tab_v):
    wid = lax.axis_index("s") * NC + lax.axis_index("c")
    pltpu.sync_copy(tab_hbm, tab_v)
    pltpu.sync_copy(gb_hbm, gb_v)
    lanes = lax.iota(jnp.int32, L)
    m_head = lanes < 7
    m_tail = lanes > 0
    zeros_i = jnp.zeros((L,), jnp.int32)
    last_i = jnp.full((L,), L - 1, jnp.int32)
    fz = jnp.zeros((L,), jnp.float32)

    gH = gb_v[pl.ds(0, L)]
    bH = gb_v[pl.ds(D_OUT, L)]
    gA = [gb_v[pl.ds(7 + 16 * m, L)] for m in range(7)]
    bA = [gb_v[pl.ds(D_OUT + 7 + 16 * m, L)] for m in range(7)]
    gT = gb_v[pl.ds(118, L)]
    bT = gb_v[pl.ds(D_OUT + 118, L)]

    def chunk_body(ci, carry):
        row0 = wid * RW + ci * CH
        pltpu.sync_copy(x_hbm.at[pl.ds(row0, CH)], x_v)

        @plsc.parallel_loop(0, CH, unroll=2)
        def rowbody(r):
            la = [x_v[r, pl.ds(16 * m + 1, L)] for m in range(7)]
            lb = x_v[r, pl.ds(112, L)]
            xv0 = x_v[r, pl.ds(0, L)]
            eib = xv0.astype(jnp.int32).at[zeros_i].get(
                mode="promise_in_bounds")
            tix = jnp.where(m_head, (eib + 1) * 7 + lanes, 0)
            ev = plsc.load_gather(tab_v, [tix], mask=m_head)
            ev = jnp.where(m_head, ev, fz)
            lbm = jnp.where(m_tail, lb, fz)
            s8 = ev + lbm
            q8 = ev * ev + lbm * lbm
            for m in range(7):
                s8 = s8 + la[m]
                q8 = q8 + la[m] * la[m]
            tot_s = plsc.cumsum(s8).at[last_i].get(mode="promise_in_bounds")
            tot_q = plsc.cumsum(q8).at[last_i].get(mode="promise_in_bounds")
            mean = tot_s * INV_D
            var = tot_q * INV_D - mean * mean
            rstd = _rsqrt_nr(var + 1e-12)
            hv = (ev - mean) * rstd * gH + bH
            plsc.store_scatter(o_v, [r + zeros_i, lanes], hv, mask=m_head)
            for m in range(7):
                o_v[r, pl.ds(7 + 16 * m, L)] = (la[m] - mean) * rstd * gA[m] + bA[m]
            o_v[r, pl.ds(118, L)] = (lb - mean) * rstd * gT + bT

        pltpu.sync_copy(o_v, out_hbm.at[pl.ds(row0, CH)])
        return carry

    lax.fori_loop(0, NCHUNK, chunk_body, 0)


def kernel(x, table, gamma, beta):
    gb = jnp.concatenate([gamma, beta]).astype(jnp.float32)
    tab = jnp.pad(table.astype(jnp.float32).reshape(-1), (0, 23))
    mesh = plsc.VectorSubcoreMesh(core_axis_name="c", subcore_axis_name="s")
    f = pl.kernel(
        _body,
        out_type=jax.ShapeDtypeStruct((B, D_OUT), jnp.float32),
        mesh=mesh,
        compiler_params=pltpu.CompilerParams(needs_layout_passes=False),
        scratch_types=[
            pltpu.VMEM((CH, D_IN), jnp.float32),
            pltpu.VMEM((CH, D_OUT), jnp.float32),
            pltpu.VMEM((2 * D_OUT,), jnp.float32),
            pltpu.VMEM((72,), jnp.float32),
        ],
    )
    return f(x, gb, tab)
